# Initial kernel scaffold; baseline (speedup 1.0000x reference)
import jax
import jax.numpy as jnp
from jax.experimental import pallas as pl


def kernel(z_fast, z_slow, cb_syn, cb_sem, Wg_syn, bg_syn, Wg_sem, bg_sem):
    raise NotImplementedError("write your pallas kernel here")


# speedup vs baseline: 1.5386x; 1.5386x over previous
import jax
import jax.numpy as jnp
from jax.experimental import pallas as pl
from jax.experimental.pallas import tpu as pltpu

GRAPH_BIAS_SCALE = 0.8
CONTEXT_GATE_STRENGTH = 2.0
COMMITMENT_COST = 0.25

N = 8192
D = 256
BN = 256


def _vq_block(z, cb, w, b):
    zn = jnp.sum(z * z, axis=1, keepdims=True)
    cn = jnp.sum(cb * cb, axis=1)
    zc = jax.lax.dot_general(z, cb, (((1,), (1,)), ((), ())),
                             preferred_element_type=jnp.float32)
    logits = jnp.dot(z, w, preferred_element_type=jnp.float32) + b
    m = jnp.max(logits, axis=1, keepdims=True)
    e = jnp.exp(logits - m)
    bias = e / jnp.sum(e, axis=1, keepdims=True)
    d = (zn + cn - 2.0 * zc) - CONTEXT_GATE_STRENGTH * bias
    idx = jnp.argmin(d, axis=1).astype(jnp.int32)
    k = cb.shape[0]
    onehot = (jax.lax.broadcasted_iota(jnp.int32, (z.shape[0], k), 1)
              == idx[:, None]).astype(jnp.float32)
    zq = jnp.dot(onehot, cb, preferred_element_type=jnp.float32)
    diff = zq - z
    return zq, idx, jnp.sum(diff * diff)


def _body(zf_ref, zs_ref, cbsyn_ref, cbsem_ref, wsyn_ref, bsyn_ref,
          wsem_ref, bsem_ref,
          zqsyn_ref, zqsem_ref, idxsyn_ref, idxsem_ref, loss_ref):
    step = pl.program_id(0)
    nsteps = pl.num_programs(0)

    @pl.when(step == 0)
    def _():
        loss_ref[0, 0] = 0.0

    zq_s, idx_s, ss_s = _vq_block(zf_ref[...], cbsyn_ref[...],
                                  wsyn_ref[...], bsyn_ref[...])
    zqsyn_ref[...] = zq_s
    idxsyn_ref[...] = idx_s

    zq_m, idx_m, ss_m = _vq_block(zs_ref[...], cbsem_ref[...],
                                  wsem_ref[...], bsem_ref[...])
    zqsem_ref[...] = zq_m
    idxsem_ref[...] = idx_m

    loss_ref[0, 0] += ss_s + ss_m

    @pl.when(step == nsteps - 1)
    def _():
        loss_ref[0, 0] = loss_ref[0, 0] * ((1.0 + COMMITMENT_COST) / (N * D))


def kernel(z_fast, z_slow, cb_syn, cb_sem, Wg_syn, bg_syn, Wg_sem, bg_sem):
    n_syn = cb_syn.shape[0]
    n_sem = cb_sem.shape[0]
    grid = (N // BN,)

    row_spec = pl.BlockSpec((BN, D), lambda i: (i, 0))
    full = lambda shape: pl.BlockSpec(shape, lambda i: (0,) * len(shape))

    out_shapes = (
        jax.ShapeDtypeStruct((N, D), jnp.float32),
        jax.ShapeDtypeStruct((N, D), jnp.float32),
        jax.ShapeDtypeStruct((N,), jnp.int32),
        jax.ShapeDtypeStruct((N,), jnp.int32),
        jax.ShapeDtypeStruct((1, 1), jnp.float32),
    )
    out_specs = (
        row_spec,
        row_spec,
        pl.BlockSpec((BN,), lambda i: (i,)),
        pl.BlockSpec((BN,), lambda i: (i,)),
        pl.BlockSpec((1, 1), lambda i: (0, 0), memory_space=pltpu.SMEM),
    )
    in_specs = [
        row_spec,
        row_spec,
        full((n_syn, D)),
        full((n_sem, D)),
        full((D, n_syn)),
        full((1, n_syn)),
        full((D, n_sem)),
        full((1, n_sem)),
    ]

    zq_syn, zq_sem, idx_syn, idx_sem, loss = pl.pallas_call(
        _body,
        grid=grid,
        in_specs=in_specs,
        out_specs=out_specs,
        out_shape=out_shapes,
    )(z_fast, z_slow, cb_syn, cb_sem, Wg_syn, bg_syn.reshape(1, n_syn),
      Wg_sem, bg_sem.reshape(1, n_sem))

    half = D // 2
    zqc_syn = jax.lax.complex(zq_syn[:, :half], zq_syn[:, half:])
    zqc_sem = jax.lax.complex(zq_sem[:, :half], zq_sem[:, half:])
    return (zqc_syn, zqc_sem, loss[0, 0], (idx_syn, idx_sem))
